# baseline (speedup 1.0000x reference)
import functools

import jax
import jax.numpy as jnp
from jax import lax
from jax.experimental import pallas as pl
from jax.experimental.layout import Layout, with_layout_constraint
from jax.experimental.pallas import tpu as pltpu
from jax.experimental.pallas import tpu_sc as plsc

B = 4096
L = 200
D = 32
NUM_EMB = 1_000_000
RW = 128
EPR = RW // D

_info = plsc.get_sparse_core_info()
NC, NS, NL = _info.num_cores, _info.num_subcores, _info.num_lanes
NW = NC * NS
BPW = B // NW
IPW = BPW * L
NBUF = 2
UNROLL = 8


def _make_kernel():
    mesh = plsc.VectorSubcoreMesh(core_axis_name="c", subcore_axis_name="s")

    @functools.partial(
        pl.kernel,
        mesh=mesh,
        out_type=jax.ShapeDtypeStruct((B, D), jnp.float32),
        compiler_params=pltpu.CompilerParams(use_tc_tiling_on_sc=False),
        scratch_types=[
            pltpu.VMEM((IPW,), jnp.int32),
            pltpu.VMEM((IPW + NL,), jnp.int32),
            [pltpu.VMEM((L, RW), jnp.float32) for _ in range(NBUF)],
            pltpu.VMEM((BPW, D), jnp.float32),
            [pltpu.SemaphoreType.DMA for _ in range(NBUF)],
        ],
    )
    def emb_pool(x_hbm, t_hbm, out_hbm, idx_v, sub_v, bufs, out_v, sems):
        wid = lax.axis_index("s") * NC + lax.axis_index("c")
        pltpu.sync_copy(x_hbm.at[pl.ds(wid * IPW, IPW)], idx_v)

        def xform(k, _):
            sl = pl.ds(k * NL, NL)
            v = idx_v[sl]
            sub_v[sl] = (v & (EPR - 1)) << 5
            idx_v[sl] = lax.shift_right_logical(v, 2)
            return _

        lax.fori_loop(0, IPW // NL, xform, 0)

        def gather_desc(b, j):
            off = pl.multiple_of(b * L, 8)
            return pltpu.make_async_copy(
                t_hbm.at[idx_v.at[pl.ds(off, L)]], bufs[j], sems[j]
            )

        def process(b, j):
            gather_desc(b, j).wait()
            buf = bufs[j]
            base = b * L

            def acc_body(k, accs):
                a0, a1, c0, c1 = accs
                l0 = k * UNROLL
                subs = sub_v[pl.ds(base + l0, NL)]
                for u in range(UNROLL):
                    l = l0 + u
                    s = subs[u]
                    r0 = buf[l, pl.ds(s, NL)]
                    r1 = buf[l, pl.ds(s + NL, NL)]
                    if u % 2 == 0:
                        a0 = a0 + r0
                        a1 = a1 + r1
                    else:
                        c0 = c0 + r0
                        c1 = c1 + r1
                return a0, a1, c0, c1

            z = jnp.zeros((NL,), jnp.float32)
            a0, a1, c0, c1 = lax.fori_loop(0, L // UNROLL, acc_body,
                                           (z, z, z, z))
            scale = jnp.float32(1.0 / L)
            out_v[b, pl.ds(0, NL)] = (a0 + c0) * scale
            out_v[b, pl.ds(NL, NL)] = (a1 + c1) * scale

        for j in range(NBUF):
            gather_desc(j, j).start()

        def main_body(i, carry):
            for j in range(NBUF):
                b = i * NBUF + j
                process(b, j)
                gather_desc(b + NBUF, j).start()
            return carry

        lax.fori_loop(0, BPW // NBUF - 1, main_body, 0)

        for j in range(NBUF):
            process(BPW - NBUF + j, j)

        pltpu.sync_copy(out_v, out_hbm.at[pl.ds(wid * BPW, BPW)])

    return emb_pool


_emb_pool = _make_kernel()


@jax.jit
def kernel(x, table):
    t128 = with_layout_constraint(
        table.reshape(NUM_EMB // EPR, RW),
        Layout(major_to_minor=(0, 1)),
    )
    return _emb_pool(x.reshape(-1), t128)


# speedup vs baseline: 1.4385x; 1.4385x over previous
import functools

import jax
import jax.numpy as jnp
from jax import lax
from jax.experimental import pallas as pl
from jax.experimental.pallas import tpu as pltpu
from jax.experimental.pallas import tpu_sc as plsc

B = 4096
L = 200
D = 32
NUM_EMB = 1_000_000
QE = 262144
RW = 128
CH = 2048

_info = plsc.get_sparse_core_info()
NC, NS, NL = _info.num_cores, _info.num_subcores, _info.num_lanes
NW = NC * NS
BPW = B // NW
IPW = BPW * L
NBUF = 2
UNROLL = 8


def _repack_body(t0, t1, t2, t3, out_ref):
    out_ref[...] = jnp.concatenate(
        [t0[...].T, t1[...].T, t2[...].T, t3[...].T], axis=1
    )


def _make_repack():
    last_block = (NUM_EMB - 1) // CH

    def _in_map(g, r):
        return (0, jnp.minimum(g * (QE // CH) + r, last_block))

    in_specs = [
        pl.BlockSpec((D, CH), functools.partial(_in_map, g)) for g in range(4)
    ]
    return pl.pallas_call(
        _repack_body,
        grid=(QE // CH,),
        in_specs=in_specs,
        out_specs=pl.BlockSpec((CH, RW), lambda r: (r, 0)),
        out_shape=jax.ShapeDtypeStruct((QE, RW), jnp.float32),
    )


_repack = _make_repack()


def _make_emb_pool():
    mesh = plsc.VectorSubcoreMesh(core_axis_name="c", subcore_axis_name="s")

    @functools.partial(
        pl.kernel,
        mesh=mesh,
        out_type=jax.ShapeDtypeStruct((B, D), jnp.float32),
        compiler_params=pltpu.CompilerParams(use_tc_tiling_on_sc=False),
        scratch_types=[
            pltpu.VMEM((IPW,), jnp.int32),
            pltpu.VMEM((IPW + NL,), jnp.int32),
            [pltpu.VMEM((L, RW), jnp.float32) for _ in range(NBUF)],
            pltpu.VMEM((BPW, D), jnp.float32),
            [pltpu.SemaphoreType.DMA for _ in range(NBUF)],
        ],
    )
    def emb_pool(x_hbm, t_hbm, out_hbm, idx_v, sub_v, bufs, out_v, sems):
        wid = lax.axis_index("s") * NC + lax.axis_index("c")
        pltpu.sync_copy(x_hbm.at[pl.ds(wid * IPW, IPW)], idx_v)

        def xform(k, _):
            sl = pl.ds(k * NL, NL)
            v = idx_v[sl]
            sub_v[sl] = lax.shift_right_logical(v, 18) << 5
            idx_v[sl] = v & (QE - 1)
            return _

        lax.fori_loop(0, IPW // NL, xform, 0)

        def gather_desc(b, j):
            off = pl.multiple_of(b * L, 8)
            return pltpu.make_async_copy(
                t_hbm.at[idx_v.at[pl.ds(off, L)]], bufs[j], sems[j]
            )

        def process(b, j):
            gather_desc(b, j).wait()
            buf = bufs[j]
            base = b * L

            def acc_body(k, accs):
                a0, a1, c0, c1 = accs
                l0 = k * UNROLL
                subs = sub_v[pl.ds(base + l0, NL)]
                for u in range(UNROLL):
                    l = l0 + u
                    s = subs[u]
                    r0 = buf[l, pl.ds(s, NL)]
                    r1 = buf[l, pl.ds(s + NL, NL)]
                    if u % 2 == 0:
                        a0 = a0 + r0
                        a1 = a1 + r1
                    else:
                        c0 = c0 + r0
                        c1 = c1 + r1
                return a0, a1, c0, c1

            z = jnp.zeros((NL,), jnp.float32)
            a0, a1, c0, c1 = lax.fori_loop(0, L // UNROLL, acc_body,
                                           (z, z, z, z))
            scale = jnp.float32(1.0 / L)
            out_v[b, pl.ds(0, NL)] = (a0 + c0) * scale
            out_v[b, pl.ds(NL, NL)] = (a1 + c1) * scale

        for j in range(NBUF):
            gather_desc(j, j).start()

        def main_body(i, carry):
            for j in range(NBUF):
                b = i * NBUF + j
                process(b, j)
                gather_desc(b + NBUF, j).start()
            return carry

        lax.fori_loop(0, BPW // NBUF - 1, main_body, 0)

        for j in range(NBUF):
            process(BPW - NBUF + j, j)

        pltpu.sync_copy(out_v, out_hbm.at[pl.ds(wid * BPW, BPW)])

    return emb_pool


_emb_pool = _make_emb_pool()


@jax.jit
def kernel(x, table):
    t_quart = _repack(table.T, table.T, table.T, table.T)
    return _emb_pool(x.reshape(-1), t_quart)


# speedup vs baseline: 1.7521x; 1.2180x over previous
import functools

import jax
import jax.numpy as jnp
from jax import lax
from jax.experimental import pallas as pl
from jax.experimental.pallas import tpu as pltpu
from jax.experimental.pallas import tpu_sc as plsc

B = 4096
L = 200
D = 32
NUM_EMB = 1_000_000
QE = 262144
NQ = 4
RW = 128
CH = 8192
SCH = 2048

_info = plsc.get_sparse_core_info()
NC, NS, NL = _info.num_cores, _info.num_subcores, _info.num_lanes
NW = NC * NS
BPW = B // NW
IPW = BPW * L
NBUF = 8
UNROLL = 8
EPW = QE // (NW // NQ)


def _repack_body(t0, t1, t2, t3, out_ref):
    out_ref[...] = jnp.concatenate(
        [t0[...].T, t1[...].T, t2[...].T, t3[...].T], axis=1
    )


def _make_repack():
    last_block = (NUM_EMB - 1) // CH

    def _in_map(g, r):
        return (0, jnp.minimum(g * (QE // CH) + r, last_block))

    in_specs = [
        pl.BlockSpec((D, CH), functools.partial(_in_map, g))
        for g in range(NQ)
    ]
    return pl.pallas_call(
        _repack_body,
        grid=(QE // CH,),
        in_specs=in_specs,
        out_specs=pl.BlockSpec((CH, RW), lambda r: (r, 0)),
        out_shape=jax.ShapeDtypeStruct((QE, RW), jnp.float32),
        compiler_params=pltpu.CompilerParams(
            fuse_transposed_lhs_in_matmul=True
        ),
    )


_repack = _make_repack()


def _make_shuffle():
    mesh = plsc.VectorSubcoreMesh(core_axis_name="c", subcore_axis_name="s")

    @functools.partial(
        pl.kernel,
        mesh=mesh,
        out_type=jax.ShapeDtypeStruct((NQ * QE, D), jnp.float32),
        compiler_params=pltpu.CompilerParams(use_tc_tiling_on_sc=False),
        scratch_types=[
            [pltpu.VMEM((SCH, D), jnp.float32) for _ in range(2)],
            [pltpu.SemaphoreType.DMA for _ in range(2)],
            [pltpu.SemaphoreType.DMA for _ in range(2)],
        ],
    )
    def shuffle(q_hbm, out_hbm, bufs, sems_in, sems_out):
        wid = lax.axis_index("s") * NC + lax.axis_index("c")
        g = lax.shift_right_logical(wid, 3)
        r0 = (wid & 7) * EPW
        col = g * D

        def desc_in(c, j):
            return pltpu.make_async_copy(
                q_hbm.at[pl.ds(r0 + c * SCH, SCH), pl.ds(col, D)],
                bufs[j], sems_in[j],
            )

        def desc_out(c, j):
            return pltpu.make_async_copy(
                bufs[j], out_hbm.at[pl.ds(g * QE + r0 + c * SCH, SCH)],
                sems_out[j],
            )

        nch = EPW // SCH

        def body(i, carry):
            for j in range(2):
                c = i * 2 + j

                @pl.when(c >= 2)
                def _(c=c, j=j):
                    desc_out(c - 2, j).wait()

                desc_in(c, j).start()
                desc_in(c, j).wait()
                desc_out(c, j).start()
            return carry

        lax.fori_loop(0, nch // 2, body, 0)
        desc_out(nch - 2, 0).wait()
        desc_out(nch - 1, 1).wait()

    return shuffle


_shuffle = _make_shuffle()


def _make_emb_pool():
    mesh = plsc.VectorSubcoreMesh(core_axis_name="c", subcore_axis_name="s")

    @functools.partial(
        pl.kernel,
        mesh=mesh,
        out_type=jax.ShapeDtypeStruct((B, D), jnp.float32),
        compiler_params=pltpu.CompilerParams(use_tc_tiling_on_sc=False),
        scratch_types=[
            pltpu.VMEM((IPW,), jnp.int32),
            [pltpu.VMEM((L, D), jnp.float32) for _ in range(NBUF)],
            pltpu.VMEM((BPW, D), jnp.float32),
            [pltpu.SemaphoreType.DMA for _ in range(NBUF)],
        ],
    )
    def emb_pool(x_hbm, t_hbm, out_hbm, idx_v, bufs, out_v, sems):
        wid = lax.axis_index("s") * NC + lax.axis_index("c")
        pltpu.sync_copy(x_hbm.at[pl.ds(wid * IPW, IPW)], idx_v)

        def gather_desc(b, j):
            off = pl.multiple_of(b * L, 8)
            return pltpu.make_async_copy(
                t_hbm.at[idx_v.at[pl.ds(off, L)]], bufs[j], sems[j]
            )

        def process(b, j):
            gather_desc(b, j).wait()
            buf = bufs[j]

            def acc_body(k, accs):
                a0, a1, c0, c1 = accs
                l0 = k * UNROLL
                for u in range(UNROLL):
                    l = l0 + u
                    r0 = buf[l, pl.ds(0, NL)]
                    r1 = buf[l, pl.ds(NL, NL)]
                    if u % 2 == 0:
                        a0 = a0 + r0
                        a1 = a1 + r1
                    else:
                        c0 = c0 + r0
                        c1 = c1 + r1
                return a0, a1, c0, c1

            z = jnp.zeros((NL,), jnp.float32)
            a0, a1, c0, c1 = lax.fori_loop(0, L // UNROLL, acc_body,
                                           (z, z, z, z))
            scale = jnp.float32(1.0 / L)
            out_v[b, pl.ds(0, NL)] = (a0 + c0) * scale
            out_v[b, pl.ds(NL, NL)] = (a1 + c1) * scale

        for j in range(NBUF):
            gather_desc(j, j).start()

        def main_body(i, carry):
            for j in range(NBUF):
                b = i * NBUF + j
                process(b, j)
                gather_desc(b + NBUF, j).start()
            return carry

        lax.fori_loop(0, BPW // NBUF - 1, main_body, 0)

        for j in range(NBUF):
            process(BPW - NBUF + j, j)

        pltpu.sync_copy(out_v, out_hbm.at[pl.ds(wid * BPW, BPW)])

    return emb_pool


_emb_pool = _make_emb_pool()


@jax.jit
def kernel(x, table):
    t_quart = _repack(table.T, table.T, table.T, table.T)
    t_rm = _shuffle(t_quart)
    return _emb_pool(x.reshape(-1), t_rm)
